# baseline (speedup 1.0000x reference)
import jax
import jax.numpy as jnp
from jax import lax
from jax.experimental import pallas as pl
from jax.experimental.pallas import tpu as pltpu
from jax.experimental.pallas import tpu_sc as plsc

N_NODES = 10000
N_EDGES = 320000
D_EDGE = 16
D_FEAT = 128
D_GLOBAL = 128
D_OUT = D_EDGE + 2 * D_FEAT + D_GLOBAL
C_RECV = D_EDGE
C_SEND = D_EDGE + D_FEAT
C_GLOB = D_EDGE + 2 * D_FEAT
L = 16

NC = 2
NS = 16
NW = NC * NS
CH = 128
NCHUNK = N_EDGES // CH
BASE_CPW = NCHUNK // NW
REMC = NCHUNK - BASE_CPW * NW
PAIRS = BASE_CPW // 2


def _edge_block(edges_t_hbm, gtile_hbm, nodes_hbm, recv_hbm, send_hbm, out_hbm,
                idx_r0, idx_r1, idx_s0, idx_s1, rows_r, rows_s, edge_v,
                band_r, band_s, gbuf,
                sem_ir, sem_is, sem_gr, sem_gs, sem_ge,
                sem_we, sem_wr, sem_ws, sem_wg):
    wid = lax.axis_index("s") * NC + lax.axis_index("c")
    nch = BASE_CPW + jnp.where(wid < REMC, 1, 0)
    sch = BASE_CPW * wid + jnp.minimum(wid, REMC)
    idx_r = (idx_r0, idx_r1)
    idx_s = (idx_s0, idx_s1)

    pltpu.sync_copy(gtile_hbm, gbuf)

    def issue_idx(c, s):
        base = (sch + c) * CH
        pltpu.async_copy(recv_hbm.at[pl.ds(base, CH)], idx_r[s], sem_ir.at[s])
        pltpu.async_copy(send_hbm.at[pl.ds(base, CH)], idx_s[s], sem_is.at[s])

    def wait_idx(s):
        pltpu.make_async_copy(recv_hbm.at[pl.ds(0, CH)], idx_r[s],
                              sem_ir.at[s]).wait()
        pltpu.make_async_copy(send_hbm.at[pl.ds(0, CH)], idx_s[s],
                              sem_is.at[s]).wait()

    def issue_gathers(c, s):
        base = (sch + c) * CH
        pltpu.async_copy(nodes_hbm.at[idx_r[s]], rows_r.at[s], sem_gr.at[s])
        pltpu.async_copy(nodes_hbm.at[idx_s[s]], rows_s.at[s], sem_gs.at[s])
        pltpu.async_copy(edges_t_hbm.at[:, pl.ds(base, CH)], edge_v.at[s],
                         sem_ge.at[s])

    def wait_gathers(s):
        pltpu.make_async_copy(nodes_hbm.at[idx_r[s]], rows_r.at[s],
                              sem_gr.at[s]).wait()
        pltpu.make_async_copy(nodes_hbm.at[idx_s[s]], rows_s.at[s],
                              sem_gs.at[s]).wait()
        pltpu.make_async_copy(edges_t_hbm.at[:, pl.ds(0, CH)], edge_v.at[s],
                              sem_ge.at[s]).wait()

    egs = [lax.iota(jnp.int32, L) + g * L for g in range(CH // L)]

    def transpose(rows, band):
        @plsc.parallel_loop(0, D_FEAT, 1, unroll=2)
        def per_feature(f):
            fidx = jnp.full((L,), 0, dtype=jnp.int32) + f
            for g in range(CH // L):
                band[f, pl.ds(g * L, L)] = plsc.load_gather(rows, [egs[g], fidx])

    def issue_writes(c, s):
        base = (sch + c) * CH
        pltpu.async_copy(edge_v.at[s],
                         out_hbm.at[pl.ds(0, D_EDGE), pl.ds(base, CH)],
                         sem_we.at[s])
        pltpu.async_copy(band_r,
                         out_hbm.at[pl.ds(C_RECV, D_FEAT), pl.ds(base, CH)],
                         sem_wr)
        pltpu.async_copy(band_s,
                         out_hbm.at[pl.ds(C_SEND, D_FEAT), pl.ds(base, CH)],
                         sem_ws)
        pltpu.async_copy(gbuf,
                         out_hbm.at[pl.ds(C_GLOB, D_GLOBAL), pl.ds(base, CH)],
                         sem_wg.at[s])

    def wait_edge_glob_writes(s):
        pltpu.make_async_copy(edge_v.at[s],
                              out_hbm.at[pl.ds(0, D_EDGE), pl.ds(0, CH)],
                              sem_we.at[s]).wait()
        pltpu.make_async_copy(gbuf,
                              out_hbm.at[pl.ds(C_GLOB, D_GLOBAL), pl.ds(0, CH)],
                              sem_wg.at[s]).wait()

    def wait_band_writes():
        pltpu.make_async_copy(band_r,
                              out_hbm.at[pl.ds(C_RECV, D_FEAT), pl.ds(0, CH)],
                              sem_wr).wait()
        pltpu.make_async_copy(band_s,
                              out_hbm.at[pl.ds(C_SEND, D_FEAT), pl.ds(0, CH)],
                              sem_ws).wait()

    def half(c, s, first=False, second=False, no_idx=False, last=False):
        wait_gathers(s)
        if not (no_idx or last):
            issue_idx(c + 2, s)
        if not first:
            wait_band_writes()
        if not (first or second):
            wait_edge_glob_writes(s)
        transpose(rows_r.at[s], band_r)
        transpose(rows_s.at[s], band_s)
        issue_writes(c, s)
        if not last:
            wait_idx(1 - s)
            issue_gathers(c + 1, 1 - s)

    issue_idx(0, 0)
    wait_idx(0)
    issue_gathers(0, 0)
    issue_idx(1, 1)

    half(0, 0, first=True)
    half(1, 1, second=True)

    def pair(j, carry):
        c = 2 * j
        half(c, 0)
        half(c + 1, 1)
        return carry

    lax.fori_loop(1, PAIRS - 1, pair, 0)

    half(BASE_CPW - 2, 0, no_idx=True)
    half(BASE_CPW - 1, 1, no_idx=True, last=True)

    def leftover(c, carry):
        issue_idx(c, 0)
        wait_idx(0)
        issue_gathers(c, 0)
        wait_gathers(0)
        wait_band_writes()
        wait_edge_glob_writes(0)
        transpose(rows_r.at[0], band_r)
        transpose(rows_s.at[0], band_s)
        issue_writes(c, 0)
        return carry

    lax.fori_loop(BASE_CPW, nch, leftover, 0)

    wait_band_writes()
    wait_edge_glob_writes(0)
    wait_edge_glob_writes(1)


@jax.jit
def _run(edges_t, g_tile, nodes_data, receivers, senders):
    mesh = plsc.VectorSubcoreMesh(core_axis_name="c", subcore_axis_name="s")
    return pl.kernel(
        _edge_block,
        mesh=mesh,
        compiler_params=pltpu.CompilerParams(needs_layout_passes=False),
        out_type=jax.ShapeDtypeStruct((D_OUT, N_EDGES), jnp.float32),
        scratch_types=[
            pltpu.VMEM((CH,), jnp.int32),
            pltpu.VMEM((CH,), jnp.int32),
            pltpu.VMEM((CH,), jnp.int32),
            pltpu.VMEM((CH,), jnp.int32),
            pltpu.VMEM((2, CH, D_FEAT), jnp.float32),
            pltpu.VMEM((2, CH, D_FEAT), jnp.float32),
            pltpu.VMEM((2, D_EDGE, CH), jnp.float32),
            pltpu.VMEM((D_FEAT, CH), jnp.float32),
            pltpu.VMEM((D_FEAT, CH), jnp.float32),
            pltpu.VMEM((D_GLOBAL, CH), jnp.float32),
            pltpu.SemaphoreType.DMA((2,)),
            pltpu.SemaphoreType.DMA((2,)),
            pltpu.SemaphoreType.DMA((2,)),
            pltpu.SemaphoreType.DMA((2,)),
            pltpu.SemaphoreType.DMA((2,)),
            pltpu.SemaphoreType.DMA((2,)),
            pltpu.SemaphoreType.DMA,
            pltpu.SemaphoreType.DMA,
            pltpu.SemaphoreType.DMA((2,)),
        ],
    )(edges_t, g_tile, nodes_data, receivers, senders)


def kernel(edges_data, nodes_data, global_data, receivers, senders):
    g_tile = jnp.broadcast_to(global_data[:, None], (D_GLOBAL, CH))
    out_t = _run(edges_data.T, g_tile, nodes_data,
                 receivers.astype(jnp.int32), senders.astype(jnp.int32))
    return out_t.T


# speedup vs baseline: 1.0569x; 1.0569x over previous
import jax
import jax.numpy as jnp
from jax import lax
from jax.experimental import pallas as pl
from jax.experimental.pallas import tpu as pltpu
from jax.experimental.pallas import tpu_sc as plsc

N_NODES = 10000
N_EDGES = 320000
D_EDGE = 16
D_FEAT = 128
D_GLOBAL = 128
D_OUT = D_EDGE + 2 * D_FEAT + D_GLOBAL
C_RECV = D_EDGE
C_SEND = D_EDGE + D_FEAT
C_GLOB = D_EDGE + 2 * D_FEAT
L = 16

NC = 2
NS = 16
NW = NC * NS
CH = 128
NCHUNK = N_EDGES // CH
BASE_CPW = NCHUNK // NW
REMC = NCHUNK - BASE_CPW * NW
PAIRS = BASE_CPW // 2


def _edge_block(edges_t_hbm, gtile_hbm, nodes_hbm, recv_hbm, send_hbm, out_hbm,
                idx_r0, idx_r1, idx_s0, idx_s1, rows_r, rows_s, edge_v,
                band_r, band_s, gbuf,
                sem_ir, sem_is, sem_gr, sem_gs, sem_ge,
                sem_we, sem_wr, sem_ws, sem_wg):
    wid = lax.axis_index("s") * NC + lax.axis_index("c")
    nch = BASE_CPW + jnp.where(wid < REMC, 1, 0)
    sch = BASE_CPW * wid + jnp.minimum(wid, REMC)
    idx_r = (idx_r0, idx_r1)
    idx_s = (idx_s0, idx_s1)

    pltpu.sync_copy(gtile_hbm, gbuf)

    def issue_idx(c, s):
        base = (sch + c) * CH
        pltpu.async_copy(recv_hbm.at[pl.ds(base, CH)], idx_r[s], sem_ir.at[s])
        pltpu.async_copy(send_hbm.at[pl.ds(base, CH)], idx_s[s], sem_is.at[s])

    def wait_idx(s):
        pltpu.make_async_copy(recv_hbm.at[pl.ds(0, CH)], idx_r[s],
                              sem_ir.at[s]).wait()
        pltpu.make_async_copy(send_hbm.at[pl.ds(0, CH)], idx_s[s],
                              sem_is.at[s]).wait()

    def issue_gathers(c, s):
        base = (sch + c) * CH
        pltpu.async_copy(nodes_hbm.at[idx_r[s]], rows_r.at[s], sem_gr.at[s])
        pltpu.async_copy(nodes_hbm.at[idx_s[s]], rows_s.at[s], sem_gs.at[s])
        pltpu.async_copy(edges_t_hbm.at[:, pl.ds(base, CH)], edge_v.at[s],
                         sem_ge.at[s])

    def wait_gathers(s):
        pltpu.make_async_copy(nodes_hbm.at[idx_r[s]], rows_r.at[s],
                              sem_gr.at[s]).wait()
        pltpu.make_async_copy(nodes_hbm.at[idx_s[s]], rows_s.at[s],
                              sem_gs.at[s]).wait()
        pltpu.make_async_copy(edges_t_hbm.at[:, pl.ds(0, CH)], edge_v.at[s],
                              sem_ge.at[s]).wait()

    egs = [lax.iota(jnp.int32, L) + g * L for g in range(CH // L)]

    def transpose(s):
        rr, rs = rows_r.at[s], rows_s.at[s]

        @plsc.parallel_loop(0, D_FEAT, 1, unroll=4)
        def per_feature(f):
            fidx = jnp.full((L,), 0, dtype=jnp.int32) + f
            for g in range(CH // L):
                band_r[f, pl.ds(g * L, L)] = plsc.load_gather(rr, [egs[g], fidx])
                band_s[f, pl.ds(g * L, L)] = plsc.load_gather(rs, [egs[g], fidx])

    def issue_writes(c, s):
        base = (sch + c) * CH
        pltpu.async_copy(edge_v.at[s],
                         out_hbm.at[pl.ds(0, D_EDGE), pl.ds(base, CH)],
                         sem_we.at[s])
        pltpu.async_copy(band_r,
                         out_hbm.at[pl.ds(C_RECV, D_FEAT), pl.ds(base, CH)],
                         sem_wr)
        pltpu.async_copy(band_s,
                         out_hbm.at[pl.ds(C_SEND, D_FEAT), pl.ds(base, CH)],
                         sem_ws)
        pltpu.async_copy(gbuf,
                         out_hbm.at[pl.ds(C_GLOB, D_GLOBAL), pl.ds(base, CH)],
                         sem_wg.at[s])

    def wait_edge_writes(s):
        pltpu.make_async_copy(edge_v.at[s],
                              out_hbm.at[pl.ds(0, D_EDGE), pl.ds(0, CH)],
                              sem_we.at[s]).wait()

    def wait_glob_writes(s):
        pltpu.make_async_copy(gbuf,
                              out_hbm.at[pl.ds(C_GLOB, D_GLOBAL), pl.ds(0, CH)],
                              sem_wg.at[s]).wait()

    def wait_band_writes():
        pltpu.make_async_copy(band_r,
                              out_hbm.at[pl.ds(C_RECV, D_FEAT), pl.ds(0, CH)],
                              sem_wr).wait()
        pltpu.make_async_copy(band_s,
                              out_hbm.at[pl.ds(C_SEND, D_FEAT), pl.ds(0, CH)],
                              sem_ws).wait()

    def half(c, s, first=False, second=False, no_idx=False, last=False):
        if not last:
            wait_idx(1 - s)
            if not first:
                wait_edge_writes(1 - s)
            issue_gathers(c + 1, 1 - s)
        wait_gathers(s)
        if not (no_idx or last):
            issue_idx(c + 2, s)
        if not first:
            wait_band_writes()
        if not (first or second):
            wait_glob_writes(s)
        transpose(s)
        issue_writes(c, s)

    issue_idx(0, 0)
    issue_idx(1, 1)
    wait_idx(0)
    issue_gathers(0, 0)

    half(0, 0, first=True)
    half(1, 1, second=True)

    def pair(j, carry):
        c = 2 * j
        half(c, 0)
        half(c + 1, 1)
        return carry

    lax.fori_loop(1, PAIRS - 1, pair, 0)

    half(BASE_CPW - 2, 0, no_idx=True)
    half(BASE_CPW - 1, 1, no_idx=True, last=True)

    def leftover(c, carry):
        issue_idx(c, 0)
        wait_idx(0)
        issue_gathers(c, 0)
        wait_gathers(0)
        wait_band_writes()
        wait_glob_writes(0)
        transpose(0)
        issue_writes(c, 0)
        wait_edge_writes(0)
        return carry

    lax.fori_loop(BASE_CPW, nch, leftover, 0)

    wait_band_writes()
    wait_edge_writes(1)
    wait_glob_writes(0)
    wait_glob_writes(1)


@jax.jit
def _run(edges_t, g_tile, nodes_data, receivers, senders):
    mesh = plsc.VectorSubcoreMesh(core_axis_name="c", subcore_axis_name="s")
    return pl.kernel(
        _edge_block,
        mesh=mesh,
        compiler_params=pltpu.CompilerParams(needs_layout_passes=False),
        out_type=jax.ShapeDtypeStruct((D_OUT, N_EDGES), jnp.float32),
        scratch_types=[
            pltpu.VMEM((CH,), jnp.int32),
            pltpu.VMEM((CH,), jnp.int32),
            pltpu.VMEM((CH,), jnp.int32),
            pltpu.VMEM((CH,), jnp.int32),
            pltpu.VMEM((2, CH, D_FEAT), jnp.float32),
            pltpu.VMEM((2, CH, D_FEAT), jnp.float32),
            pltpu.VMEM((2, D_EDGE, CH), jnp.float32),
            pltpu.VMEM((D_FEAT, CH), jnp.float32),
            pltpu.VMEM((D_FEAT, CH), jnp.float32),
            pltpu.VMEM((D_GLOBAL, CH), jnp.float32),
            pltpu.SemaphoreType.DMA((2,)),
            pltpu.SemaphoreType.DMA((2,)),
            pltpu.SemaphoreType.DMA((2,)),
            pltpu.SemaphoreType.DMA((2,)),
            pltpu.SemaphoreType.DMA((2,)),
            pltpu.SemaphoreType.DMA((2,)),
            pltpu.SemaphoreType.DMA,
            pltpu.SemaphoreType.DMA,
            pltpu.SemaphoreType.DMA((2,)),
        ],
    )(edges_t, g_tile, nodes_data, receivers, senders)


def kernel(edges_data, nodes_data, global_data, receivers, senders):
    g_tile = jnp.broadcast_to(global_data[:, None], (D_GLOBAL, CH))
    out_t = _run(edges_data.T, g_tile, nodes_data,
                 receivers.astype(jnp.int32), senders.astype(jnp.int32))
    return out_t.T


# speedup vs baseline: 1.1994x; 1.1349x over previous
import jax
import jax.numpy as jnp
from jax import lax
from jax.experimental import pallas as pl
from jax.experimental.pallas import tpu as pltpu
from jax.experimental.pallas import tpu_sc as plsc

N_NODES = 10000
N_EDGES = 320000
D_EDGE = 16
D_FEAT = 128
D_GLOBAL = 128
D_OUT = D_EDGE + 2 * D_FEAT + D_GLOBAL
C_RECV = D_EDGE
C_SEND = D_EDGE + D_FEAT
C_GLOB = D_EDGE + 2 * D_FEAT
L = 16

NC = 2
NS = 16
NW = NC * NS
E_PER_W = N_EDGES // NW
B = 40
NCHUNK = E_PER_W // B
NIN = 3
K = 2
PERIOD = 6
GROUPS = (NCHUNK - 4) // PERIOD
IDX_PAD = 10112


def _edge_block(edges_hbm, glob_hbm, nodes_hbm, recv_hbm, send_hbm, out_hbm,
                idx_r, idx_s, rows_r, rows_s, edge_v, gvec, tile,
                sem_gr, sem_gs, sem_ge, sem_out):
    wid = lax.axis_index("s") * NC + lax.axis_index("c")
    wbase = wid * E_PER_W

    pltpu.sync_copy(recv_hbm.at[pl.ds(wid * IDX_PAD, IDX_PAD)], idx_r)
    pltpu.sync_copy(send_hbm.at[pl.ds(wid * IDX_PAD, IDX_PAD)], idx_s)
    pltpu.sync_copy(glob_hbm, gvec)

    def fill_glob(r, carry):
        for b in range(2):
            for k in range(D_GLOBAL // L):
                tile[b, r, pl.ds(C_GLOB + k * L, L)] = gvec[pl.ds(k * L, L)]
        return carry

    lax.fori_loop(0, B, fill_glob, 0)

    SPLITS = ((0, 24), (24, 16))

    def issue_inputs(c, b):
        base = wbase + c * B
        for o, n in SPLITS:
            pltpu.async_copy(nodes_hbm.at[idx_r.at[pl.ds(c * B + o, n)]],
                             rows_r.at[b, pl.ds(o, n)], sem_gr.at[b])
            pltpu.async_copy(nodes_hbm.at[idx_s.at[pl.ds(c * B + o, n)]],
                             rows_s.at[b, pl.ds(o, n)], sem_gs.at[b])
        pltpu.async_copy(edges_hbm.at[pl.ds(base, B)], edge_v.at[b],
                         sem_ge.at[b])

    def wait_inputs(b):
        for o, n in SPLITS:
            pltpu.make_async_copy(nodes_hbm.at[idx_r.at[pl.ds(o, n)]],
                                  rows_r.at[b, pl.ds(o, n)],
                                  sem_gr.at[b]).wait()
            pltpu.make_async_copy(nodes_hbm.at[idx_s.at[pl.ds(o, n)]],
                                  rows_s.at[b, pl.ds(o, n)],
                                  sem_gs.at[b]).wait()
        pltpu.make_async_copy(edges_hbm.at[pl.ds(0, B)], edge_v.at[b],
                              sem_ge.at[b]).wait()

    def issue_output(c, b):
        base = wbase + c * B
        for o, n in SPLITS:
            pltpu.async_copy(tile.at[b, pl.ds(o, n)],
                             out_hbm.at[pl.ds(base + o, n)], sem_out.at[b])

    def wait_output(b):
        for o, n in SPLITS:
            pltpu.make_async_copy(tile.at[b, pl.ds(o, n)],
                                  out_hbm.at[pl.ds(o, n)],
                                  sem_out.at[b]).wait()

    def assemble(bi, bt):
        @plsc.parallel_loop(0, B, 1, unroll=4)
        def row(r):
            tile[bt, r, pl.ds(0, L)] = edge_v[bi, r, pl.ds(0, L)]
            for k in range(D_FEAT // L):
                tile[bt, r, pl.ds(C_RECV + k * L, L)] = \
                    rows_r[bi, r, pl.ds(k * L, L)]
                tile[bt, r, pl.ds(C_SEND + k * L, L)] = \
                    rows_s[bi, r, pl.ds(k * L, L)]

    def step(c, j, head=False, tail=False):
        bi, bt = j % NIN, j % 2
        if not tail:
            issue_inputs(c + K, (j + K) % NIN)
        if not head:
            wait_output(bt)
        wait_inputs(bi)
        assemble(bi, bt)
        issue_output(c, bt)

    for q in range(K):
        issue_inputs(q, q % NIN)
    for j in range(PERIOD):
        step(j, j, head=(j < 2))

    def period(g, carry):
        base = g * PERIOD
        for j in range(PERIOD):
            step(base + j, j)
        return carry

    lax.fori_loop(1, GROUPS, period, 0)

    for c in range(NCHUNK - 4, NCHUNK):
        step(c, c % PERIOD, tail=(c + K >= NCHUNK))

    wait_output(0)
    wait_output(1)


@jax.jit
def _run(edges_data, global_data, nodes_data, receivers, senders):
    mesh = plsc.VectorSubcoreMesh(core_axis_name="c", subcore_axis_name="s")
    return pl.kernel(
        _edge_block,
        mesh=mesh,
        out_type=jax.ShapeDtypeStruct((N_EDGES, D_OUT), jnp.float32),
        scratch_types=[
            pltpu.VMEM((IDX_PAD,), jnp.int32),
            pltpu.VMEM((IDX_PAD,), jnp.int32),
            pltpu.VMEM((NIN, B, D_FEAT), jnp.float32),
            pltpu.VMEM((NIN, B, D_FEAT), jnp.float32),
            pltpu.VMEM((NIN, B, D_EDGE), jnp.float32),
            pltpu.VMEM((D_GLOBAL,), jnp.float32),
            pltpu.VMEM((2, B, D_OUT), jnp.float32),
            pltpu.SemaphoreType.DMA((NIN,)),
            pltpu.SemaphoreType.DMA((NIN,)),
            pltpu.SemaphoreType.DMA((NIN,)),
            pltpu.SemaphoreType.DMA((2,)),
        ],
    )(edges_data, global_data, nodes_data, receivers, senders)


def kernel(edges_data, nodes_data, global_data, receivers, senders):
    pad = IDX_PAD - E_PER_W
    recv = jnp.pad(receivers.astype(jnp.int32).reshape(NW, E_PER_W),
                   ((0, 0), (0, pad))).reshape(NW * IDX_PAD)
    send = jnp.pad(senders.astype(jnp.int32).reshape(NW, E_PER_W),
                   ((0, 0), (0, pad))).reshape(NW * IDX_PAD)
    return _run(edges_data, global_data, nodes_data, recv, send)
